# baseline (speedup 1.0000x reference)
import functools

import jax
import jax.numpy as jnp
from jax import lax
from jax.experimental import pallas as pl
from jax.experimental.pallas import tpu as pltpu
from jax.experimental.pallas import tpu_sc as plsc

EMBED = 64
LN_EPS = 1e-12
L = 16
NW = 32
CHUNK = 640
SUB = 128
NSUB = CHUNK // SUB
NQ = EMBED // L


def _rsqrt(x):
    i = lax.bitcast_convert_type(x, jnp.int32)
    i = jnp.int32(0x5F3759DF) - (i >> 1)
    y = lax.bitcast_convert_type(i, jnp.float32)
    for _ in range(2):
        y = y * (1.5 - 0.5 * x * y * y)
    return y


def _bcast_lane(vec, k):
    return jnp.take_along_axis(vec, jnp.full((L,), k, jnp.int32), axis=0)


def _sc_body(nrows, x_hbm, table_hbm, gamma_hbm, beta_hbm, out_hbm,
             idx_all, rows0, rows1, gamma_v, beta_v,
             sem_g0, sem_g1, sem_o0, sem_o1):
    rpw = nrows // NW
    nchunk = rpw // CHUNK
    wid = lax.axis_index("s") * 2 + lax.axis_index("c")
    row0 = wid * rpw

    rows = (rows0, rows1)
    sem_g = (sem_g0, sem_g1)
    sem_o = (sem_o0, sem_o1)

    pltpu.sync_copy(gamma_hbm, gamma_v)
    pltpu.sync_copy(beta_hbm, beta_v)
    gs = [gamma_v[pl.ds(q * L, L)] for q in range(NQ)]
    bs = [beta_v[pl.ds(q * L, L)] for q in range(NQ)]

    pltpu.sync_copy(x_hbm.at[wid], idx_all)

    iota = lax.iota(jnp.int32, L)

    def fire_gather(c, b):
        for k in range(NSUB):
            pltpu.async_copy(table_hbm.at[idx_all.at[c * NSUB + k]],
                             rows[b].at[pl.ds(k * SUB, SUB)], sem_g[b])

    def wait_gather(c, b):
        for k in range(NSUB):
            pltpu.make_async_copy(table_hbm.at[idx_all.at[c * NSUB + k]],
                                  rows[b].at[pl.ds(k * SUB, SUB)],
                                  sem_g[b]).wait()

    def fire_out(c, b):
        r0 = row0 + c * CHUNK
        pltpu.async_copy(rows[b], out_hbm.at[pl.ds(r0, CHUNK)], sem_o[b])

    def wait_out(c, b):
        r0 = row0 + c * CHUNK
        pltpu.make_async_copy(rows[b], out_hbm.at[pl.ds(r0, CHUNK)],
                              sem_o[b]).wait()

    def compute(b):
        rows_v = rows[b]

        @plsc.parallel_loop(0, CHUNK // L, unroll=4)
        def group_body(grp):
            base = grp * L
            row_ids = base + iota
            sa = [jnp.zeros((L,), jnp.float32) for _ in range(4)]
            sq = [jnp.zeros((L,), jnp.float32) for _ in range(4)]
            for j in range(EMBED):
                q, r = divmod(j, L)
                col = (q * L) + (iota ^ r)
                v = plsc.load_gather(rows_v, [row_ids, col])
                sa[j & 3] = sa[j & 3] + v
                sq[j & 3] = sq[j & 3] + v * v
            s = (sa[0] + sa[1]) + (sa[2] + sa[3])
            ss = (sq[0] + sq[1]) + (sq[2] + sq[3])
            mean = s * (1.0 / EMBED)
            var = ss * (1.0 / EMBED) - mean * mean
            rstd = _rsqrt(var + LN_EPS)
            for k in range(L):
                r = base + k
                mb = _bcast_lane(mean, k)
                rb = _bcast_lane(rstd, k)
                for q in range(NQ):
                    v = rows_v[r, pl.ds(q * L, L)]
                    rows_v[r, pl.ds(q * L, L)] = (v - mb) * rb * gs[q] + bs[q]

    fire_gather(0, 0)

    def super_body(go, _):
        for phase in range(2):
            c = 2 * go + phase
            b = phase
            nb = 1 - phase
            wait_gather(c, b)

            @pl.when(c + 1 < nchunk)
            def _():
                @pl.when(c >= 1)
                def _():
                    wait_out(c - 1, nb)

                fire_gather(c + 1, nb)

            compute(b)
            fire_out(c, b)
        return 0

    lax.fori_loop(0, nchunk // 2, super_body, 0)
    wait_out(nchunk - 2, 0)
    wait_out(nchunk - 1, 1)


@functools.partial(jax.jit, static_argnames=("nrows",))
def _run(x3d, table, gamma, beta, nrows):
    mesh = plsc.VectorSubcoreMesh(core_axis_name="c", subcore_axis_name="s")
    rpw = nrows // NW
    kfn = pl.kernel(
        functools.partial(_sc_body, nrows),
        mesh=mesh,
        compiler_params=pltpu.CompilerParams(
            needs_layout_passes=False, use_tc_tiling_on_sc=False,
            skip_device_barrier=True),
        out_type=jax.ShapeDtypeStruct((nrows, EMBED), jnp.float32),
        scratch_types=[
            pltpu.VMEM((rpw // SUB, SUB), jnp.int32),
            pltpu.VMEM((CHUNK, EMBED), jnp.float32),
            pltpu.VMEM((CHUNK, EMBED), jnp.float32),
            pltpu.VMEM((EMBED,), jnp.float32),
            pltpu.VMEM((EMBED,), jnp.float32),
            pltpu.SemaphoreType.DMA,
            pltpu.SemaphoreType.DMA,
            pltpu.SemaphoreType.DMA,
            pltpu.SemaphoreType.DMA,
        ],
    )
    return kfn(x3d, table, gamma, beta)


def kernel(x, table, gamma, beta):
    B, S = x.shape
    nrows = B * S
    rpw = nrows // NW
    assert nrows % (NW * CHUNK) == 0 and (rpw // CHUNK) % 2 == 0
    x3d = x.reshape(NW, rpw // SUB, SUB)
    out = _run(x3d, table, gamma, beta, nrows)
    return out.reshape(B, S, EMBED)


# speedup vs baseline: 1.1967x; 1.1967x over previous
import functools

import jax
import jax.numpy as jnp
from jax import lax
from jax.experimental import pallas as pl
from jax.experimental.pallas import tpu as pltpu
from jax.experimental.pallas import tpu_sc as plsc

EMBED = 64
LN_EPS = 1e-12
L = 16
NW = 32
CHUNK = 640
SUB = 128
NSUB = CHUNK // SUB
NQ = EMBED // L


def _rsqrt(x):
    i = lax.bitcast_convert_type(x, jnp.int32)
    i = jnp.int32(0x5F3759DF) - (i >> 1)
    y = lax.bitcast_convert_type(i, jnp.float32)
    for _ in range(2):
        y = y * (1.5 - 0.5 * x * y * y)
    return y


def _bcast_lane(vec, k):
    return jnp.take_along_axis(vec, jnp.full((L,), k, jnp.int32), axis=0)


def _sc_body(nrows, x_hbm, table_hbm, gamma_hbm, beta_hbm, out_hbm,
             idx_all, rows0, rows1, gamma_v, beta_v,
             sem_g0, sem_g1, sem_o0, sem_o1):
    rpw = nrows // NW
    nchunk = rpw // CHUNK
    wid = lax.axis_index("s") * 2 + lax.axis_index("c")
    row0 = wid * rpw

    rows = (rows0, rows1)
    sem_g = (sem_g0, sem_g1)
    sem_o = (sem_o0, sem_o1)

    pltpu.sync_copy(gamma_hbm, gamma_v)
    pltpu.sync_copy(beta_hbm, beta_v)
    gs = [gamma_v[pl.ds(q * L, L)] for q in range(NQ)]
    bs = [beta_v[pl.ds(q * L, L)] for q in range(NQ)]

    pltpu.sync_copy(x_hbm.at[wid], idx_all)

    iota = lax.iota(jnp.int32, L)

    def fire_gather(c, b):
        for k in range(NSUB):
            pltpu.async_copy(table_hbm.at[idx_all.at[c * NSUB + k]],
                             rows[b].at[pl.ds(k * SUB, SUB)], sem_g[b])

    def wait_gather(c, b):
        for k in range(NSUB):
            pltpu.make_async_copy(table_hbm.at[idx_all.at[c * NSUB + k]],
                                  rows[b].at[pl.ds(k * SUB, SUB)],
                                  sem_g[b]).wait()

    def fire_out(c, b):
        r0 = row0 + c * CHUNK
        pltpu.async_copy(rows[b], out_hbm.at[pl.ds(r0, CHUNK)], sem_o[b])

    def wait_out(c, b):
        r0 = row0 + c * CHUNK
        pltpu.make_async_copy(rows[b], out_hbm.at[pl.ds(r0, CHUNK)],
                              sem_o[b]).wait()

    def compute(b):
        rows_v = rows[b]

        @plsc.parallel_loop(0, CHUNK // L, unroll=2)
        def group_body(grp):
            base = grp * L
            row_ids = base + iota
            sa = [jnp.zeros((L,), jnp.float32) for _ in range(4)]
            sq = [jnp.zeros((L,), jnp.float32) for _ in range(4)]
            for j in range(EMBED):
                q, r = divmod(j, L)
                col = (q * L) + (iota ^ r)
                v = plsc.load_gather(rows_v, [row_ids, col])
                sa[j & 3] = sa[j & 3] + v
                sq[j & 3] = sq[j & 3] + v * v
            s = (sa[0] + sa[1]) + (sa[2] + sa[3])
            ss = (sq[0] + sq[1]) + (sq[2] + sq[3])
            mean = s * (1.0 / EMBED)
            var = ss * (1.0 / EMBED) - mean * mean
            rstd = _rsqrt(var + LN_EPS)
            for k in range(L):
                r = base + k
                mb = _bcast_lane(mean, k)
                rb = _bcast_lane(rstd, k)
                for q in range(NQ):
                    v = rows_v[r, pl.ds(q * L, L)]
                    rows_v[r, pl.ds(q * L, L)] = (v - mb) * rb * gs[q] + bs[q]

    fire_gather(0, 0)

    def super_body(go, _):
        for phase in range(2):
            c = 2 * go + phase
            b = phase
            nb = 1 - phase
            wait_gather(c, b)

            @pl.when(c + 1 < nchunk)
            def _():
                @pl.when(c >= 1)
                def _():
                    wait_out(c - 1, nb)

                fire_gather(c + 1, nb)

            compute(b)
            fire_out(c, b)
        return 0

    lax.fori_loop(0, nchunk // 2, super_body, 0)
    wait_out(nchunk - 2, 0)
    wait_out(nchunk - 1, 1)


@functools.partial(jax.jit, static_argnames=("nrows",))
def _run(x3d, table, gamma, beta, nrows):
    mesh = plsc.VectorSubcoreMesh(core_axis_name="c", subcore_axis_name="s")
    rpw = nrows // NW
    kfn = pl.kernel(
        functools.partial(_sc_body, nrows),
        mesh=mesh,
        compiler_params=pltpu.CompilerParams(
            needs_layout_passes=False, use_tc_tiling_on_sc=False,
            skip_device_barrier=True),
        out_type=jax.ShapeDtypeStruct((nrows, EMBED), jnp.float32),
        scratch_types=[
            pltpu.VMEM((rpw // SUB, SUB), jnp.int32),
            pltpu.VMEM((CHUNK, EMBED), jnp.float32),
            pltpu.VMEM((CHUNK, EMBED), jnp.float32),
            pltpu.VMEM((EMBED,), jnp.float32),
            pltpu.VMEM((EMBED,), jnp.float32),
            pltpu.SemaphoreType.DMA,
            pltpu.SemaphoreType.DMA,
            pltpu.SemaphoreType.DMA,
            pltpu.SemaphoreType.DMA,
        ],
    )
    return kfn(x3d, table, gamma, beta)


def kernel(x, table, gamma, beta):
    B, S = x.shape
    nrows = B * S
    rpw = nrows // NW
    assert nrows % (NW * CHUNK) == 0 and (rpw // CHUNK) % 2 == 0
    x3d = x.reshape(NW, rpw // SUB, SUB)
    out = _run(x3d, table, gamma, beta, nrows)
    return out.reshape(B, S, EMBED)
